# Initial kernel scaffold; baseline (speedup 1.0000x reference)
import jax
import jax.numpy as jnp
from jax.experimental import pallas as pl


def kernel(tokens, arc_A, arc_start, arc_stride):
    raise NotImplementedError("write your pallas kernel here")


# speedup vs baseline: 5.3632x; 5.3632x over previous
import jax
import jax.numpy as jnp
from jax import lax
from jax.experimental import pallas as pl
from jax.experimental.pallas import tpu as pltpu
from jax.experimental.pallas import tpu_sc as plsc

NC, NS, L = 2, 16, 16
NW = NC * NS
ROWS, COLS = 16384, 200
N_TOK = ROWS * COLS
NT = N_TOK // NW
CH = 4096
NCH = NT // CH


def _table_body(a_ref, s_ref, d_ref, o_ref):
    lane = lax.broadcasted_iota(jnp.int32, (8, 128), 1)
    d = (lane >> 1).astype(jnp.float32)
    ang = s_ref[0] + d * d_ref[0]
    o_ref[...] = a_ref[0] * jnp.where(lane & 1 == 0, jnp.cos(ang), jnp.sin(ang))


def _build_table(arc_A, arc_start, arc_stride):
    out = pl.pallas_call(
        _table_body,
        out_shape=jax.ShapeDtypeStruct((8, 128), jnp.float32),
        in_specs=[pl.BlockSpec(memory_space=pltpu.SMEM)] * 3,
    )(arc_A.reshape(1), arc_start.reshape(1), arc_stride.reshape(1))
    return out[0, :32]


def _sc_body(table_hbm, tok_hbm, out_hbm, table_v, tok_v, out_v):
    wid = lax.axis_index("s") * NC + lax.axis_index("c")
    base = wid * NT
    pltpu.sync_copy(table_hbm, table_v)
    iota = lax.iota(jnp.int32, L)
    half = iota >> 1
    par = iota & 1

    @pl.loop(0, NCH)
    def _chunk(g):
        tstart = base + g * CH
        pltpu.sync_copy(tok_hbm.at[pl.ds(tstart, CH)], tok_v)

        @pl.loop(0, CH // L)
        def _grp(k):
            b = k * L
            t_lo = plsc.load_gather(tok_v, [b + half])
            t_hi = plsc.load_gather(tok_v, [b + 8 + half])
            out_v[pl.ds(2 * b, L)] = plsc.load_gather(table_v, [t_lo * 2 + par])
            out_v[pl.ds(2 * b + L, L)] = plsc.load_gather(table_v, [t_hi * 2 + par])

        pltpu.sync_copy(out_v, out_hbm.at[pl.ds(2 * tstart, 2 * CH)])


def kernel(tokens, arc_A, arc_start, arc_stride):
    table = _build_table(arc_A, arc_start, arc_stride)
    sc = pl.kernel(
        _sc_body,
        out_type=jax.ShapeDtypeStruct((2 * N_TOK,), jnp.float32),
        mesh=plsc.VectorSubcoreMesh(core_axis_name="c", subcore_axis_name="s"),
        compiler_params=pltpu.CompilerParams(needs_layout_passes=False),
        scratch_types=[
            pltpu.VMEM((32,), jnp.float32),
            pltpu.VMEM((CH,), jnp.int32),
            pltpu.VMEM((2 * CH,), jnp.float32),
        ],
    )
    out = sc(table, tokens.reshape(-1))
    return out.reshape(ROWS, COLS, 2)


# speedup vs baseline: 94.5194x; 17.6236x over previous
import jax
import jax.numpy as jnp
from jax import lax
from jax.experimental import pallas as pl
from jax.experimental.pallas import tpu as pltpu
from jax.experimental.pallas import tpu_sc as plsc

NC, NS, L = 2, 16, 16
NW = NC * NS
ROWS, COLS = 16384, 200
N_TOK = ROWS * COLS
JB, IB = COLS // 8, ROWS // 128
IBB = 8
UNITS = JB * (IB // IBB)
U_TOK = IBB * 1024
U_OUT = 2 * U_TOK


def _table_body(a_ref, s_ref, d_ref, o_ref):
    lane = lax.broadcasted_iota(jnp.int32, (8, 128), 1)
    d = (lane >> 1).astype(jnp.float32)
    ang = s_ref[0] + d * d_ref[0]
    o_ref[...] = a_ref[0] * jnp.where(lane & 1 == 0, jnp.cos(ang), jnp.sin(ang))


def _build_table(arc_A, arc_start, arc_stride):
    out = pl.pallas_call(
        _table_body,
        out_shape=jax.ShapeDtypeStruct((8, 128), jnp.float32),
        in_specs=[pl.BlockSpec(memory_space=pltpu.SMEM)] * 3,
    )(arc_A.reshape(1), arc_start.reshape(1), arc_stride.reshape(1))
    return out[0, :32]


def _sc_body(table_hbm, tok_hbm, out_hbm, table_v, tok_v, out_v):
    wid = lax.axis_index("s") * NC + lax.axis_index("c")
    pltpu.sync_copy(table_hbm, table_v)

    @pl.loop(wid, UNITS, step=NW)
    def _unit(u):
        jb = u // (IB // IBB)
        ibb = u % (IB // IBB)
        pltpu.sync_copy(tok_hbm.at[pl.ds(jb * 131072 + ibb * U_TOK, U_TOK)], tok_v)

        @pl.loop(0, 64)
        def _jp(jp):
            jr = jp >> 3
            ibl = jp & 7
            src = ibl * 1024 + jr * 128
            dst = jr * 2048 + ibl * 256

            @pl.loop(0, 8, unroll=4)
            def _vec(s):
                t2 = tok_v[pl.ds(src + s * 16, 16)] * 2
                out_v[pl.ds(dst + s * 16, 16)] = plsc.load_gather(table_v, [t2])
                out_v[pl.ds(dst + 128 + s * 16, 16)] = plsc.load_gather(
                    table_v, [t2 + 1])

        @pl.loop(0, 8)
        def _wb(jr):
            pltpu.sync_copy(
                out_v.at[pl.ds(jr * 2048, 2048)],
                out_hbm.at[pl.ds((jb * 8 + jr) * 32768 + ibb * 2048, 2048)])


def kernel(tokens, arc_A, arc_start, arc_stride):
    table = _build_table(arc_A, arc_start, arc_stride)
    tok_flat = (
        tokens.T.reshape(JB, 8, IB, 128).transpose(0, 2, 1, 3).reshape(-1))
    sc = pl.kernel(
        _sc_body,
        out_type=jax.ShapeDtypeStruct((2 * N_TOK,), jnp.float32),
        mesh=plsc.VectorSubcoreMesh(core_axis_name="c", subcore_axis_name="s"),
        compiler_params=pltpu.CompilerParams(needs_layout_passes=False),
        scratch_types=[
            pltpu.VMEM((32,), jnp.float32),
            pltpu.VMEM((U_TOK,), jnp.int32),
            pltpu.VMEM((U_OUT,), jnp.float32),
        ],
    )
    out = sc(table, tok_flat)
    return out.reshape(COLS, IB, 2, 128).transpose(1, 3, 0, 2).reshape(
        ROWS, COLS, 2)


# speedup vs baseline: 129.2628x; 1.3676x over previous
import jax
import jax.numpy as jnp
from jax import lax
from jax.experimental import pallas as pl
from jax.experimental.pallas import tpu as pltpu
from jax.experimental.pallas import tpu_sc as plsc

NC, NS, L = 2, 16, 16
NW = NC * NS
ROWS, COLS = 16384, 200
N_TOK = ROWS * COLS
JB, IB = COLS // 8, ROWS // 128
IBB = 4
UNITS = JB * (IB // IBB)
UPW = UNITS // NW
U_TOK = IBB * 1024
U_OUT = 2 * U_TOK


def _table_body(a_ref, s_ref, d_ref, o_ref):
    lane = lax.broadcasted_iota(jnp.int32, (8, 128), 1)
    d = (lane & 15).astype(jnp.float32)
    ang = s_ref[0] + d * d_ref[0]
    o_ref[...] = a_ref[0] * jnp.where(lane & 16 == 0, jnp.cos(ang), jnp.sin(ang))


def _build_table(arc_A, arc_start, arc_stride):
    out = pl.pallas_call(
        _table_body,
        out_shape=jax.ShapeDtypeStruct((8, 128), jnp.float32),
        in_specs=[pl.BlockSpec(memory_space=pltpu.SMEM)] * 3,
    )(arc_A.reshape(1), arc_start.reshape(1), arc_stride.reshape(1))
    return out[0, :32]


def _sc_body(table_hbm, tok_hbm, out_hbm, table_v,
             tok_v0, tok_v1, out_v0, out_v1, isem0, isem1, osem0, osem1):
    toks, outs = (tok_v0, tok_v1), (out_v0, out_v1)
    isems, osems = (isem0, isem1), (osem0, osem1)
    wid = lax.axis_index("s") * NC + lax.axis_index("c")
    base = wid * UPW
    pltpu.sync_copy(table_hbm, table_v)

    def start_in(n, b):
        u = base + n
        jb = u >> 5
        ibb = u & 31
        pltpu.async_copy(
            tok_hbm.at[pl.ds(jb * 131072 + ibb * U_TOK, U_TOK)], toks[b],
            isems[b])

    def drain_in(b):
        pltpu.make_async_copy(
            tok_hbm.at[pl.ds(0, U_TOK)], toks[b], isems[b]).wait()

    def start_out(n, b):
        u = base + n
        jb = u >> 5
        ibb = u & 31
        for jr in range(8):
            pltpu.async_copy(
                outs[b].at[pl.ds(jr * 1024, 1024)],
                out_hbm.at[pl.ds((jb * 8 + jr) * 32768 + ibb * 1024, 1024)],
                osems[b])

    def drain_out(b):
        pltpu.make_async_copy(
            out_hbm.at[pl.ds(0, U_OUT)], outs[b], osems[b]).wait()

    def compute(b):
        tok_ref, out_ref = toks[b], outs[b]

        @pl.loop(0, 32)
        def _jp(jp):
            jr = jp >> 2
            ibl = jp & 3
            src = ibl * 1024 + jr * 128
            dst = jr * 1024 + ibl * 256

            @pl.loop(0, 8, unroll=8)
            def _vec(s):
                t = tok_ref[pl.ds(src + s * 16, 16)]
                out_ref[pl.ds(dst + s * 16, 16)] = plsc.load_gather(
                    table_v, [t])
                out_ref[pl.ds(dst + 128 + s * 16, 16)] = plsc.load_gather(
                    table_v, [t + 16])

    start_in(0, 0)
    start_in(1, 1)

    @pl.loop(0, UPW - 1, step=2)
    def _g(g):
        for b in range(2):
            n = g + b
            drain_in(b)

            @pl.when(n >= 2)
            def _do(b=b):
                drain_out(b)

            compute(b)
            start_out(n, b)

            @pl.when(n + 2 < UPW)
            def _di(n=n, b=b):
                start_in(n + 2, b)

    drain_in(0)
    drain_out(0)
    compute(0)
    start_out(UPW - 1, 0)
    drain_out(1)
    drain_out(0)


def kernel(tokens, arc_A, arc_start, arc_stride):
    table = _build_table(arc_A, arc_start, arc_stride)
    tok_flat = (
        tokens.T.reshape(JB, 8, IB, 128).transpose(0, 2, 1, 3).reshape(-1))
    sc = pl.kernel(
        _sc_body,
        out_type=jax.ShapeDtypeStruct((2 * N_TOK,), jnp.float32),
        mesh=plsc.VectorSubcoreMesh(core_axis_name="c", subcore_axis_name="s"),
        compiler_params=pltpu.CompilerParams(needs_layout_passes=False),
        scratch_types=[
            pltpu.VMEM((32,), jnp.float32),
            pltpu.VMEM((U_TOK,), jnp.int32),
            pltpu.VMEM((U_TOK,), jnp.int32),
            pltpu.VMEM((U_OUT,), jnp.float32),
            pltpu.VMEM((U_OUT,), jnp.float32),
            pltpu.SemaphoreType.DMA,
            pltpu.SemaphoreType.DMA,
            pltpu.SemaphoreType.DMA,
            pltpu.SemaphoreType.DMA,
        ],
    )
    out = sc(table, tok_flat)
    return out.reshape(COLS, IB, 2, 128).transpose(1, 3, 0, 2).reshape(
        ROWS, COLS, 2)


# speedup vs baseline: 306.1248x; 2.3682x over previous
import jax
import jax.numpy as jnp
from jax import lax
from jax.experimental import pallas as pl
from jax.experimental.pallas import tpu as pltpu
from jax.experimental.pallas import tpu_sc as plsc

NC, NS, L = 2, 16, 16
NW = NC * NS
ROWS, COLS = 16384, 200
N_TOK = ROWS * COLS
JB, IB = COLS // 8, ROWS // 128
IBB = 4
UNITS = JB * (IB // IBB)
UPW = UNITS // NW
U_TOK = IBB * 1024
U_OUT = 2 * U_TOK

_PIO2_HI = 1.5707855224609375
_PIO2_LO = 1.0804334123550503e-05
_TWO_OVER_PI = 0.6366197723675814


def _sincos_table(scal_v, table_v):
    idx0 = jnp.zeros((L,), jnp.int32)
    a = plsc.load_gather(scal_v, [idx0 + 1])
    start = plsc.load_gather(scal_v, [idx0 + 2])
    stride = plsc.load_gather(scal_v, [idx0 + 3])
    d = lax.iota(jnp.int32, L).astype(jnp.float32)
    ang = start + d * stride
    kf = ang * _TWO_OVER_PI
    ki = jnp.where(kf >= 0, kf + 0.5, kf - 0.5).astype(jnp.int32)
    kx = ki.astype(jnp.float32)
    r = (ang - kx * _PIO2_HI) - kx * _PIO2_LO
    q = ki & 3
    r2 = r * r
    sp = r * (1.0 + r2 * (-1.0 / 6 + r2 * (1.0 / 120 + r2 * (-1.0 / 5040))))
    cp = 1.0 + r2 * (-1.0 / 2 + r2 * (1.0 / 24 + r2 * (
        -1.0 / 720 + r2 * (1.0 / 40320))))
    cos_v = jnp.where(q == 0, cp, jnp.where(q == 1, -sp,
                      jnp.where(q == 2, -cp, sp)))
    sin_v = jnp.where(q == 0, sp, jnp.where(q == 1, cp,
                      jnp.where(q == 2, -sp, -cp)))
    table_v[pl.ds(0, L)] = a * cos_v
    table_v[pl.ds(16, L)] = a * sin_v


def _sc_body(scal_hbm, tok_hbm, out_hbm, scal_v, table_v,
             tok_v0, tok_v1, out_v0, out_v1, isem0, isem1, osem0, osem1):
    toks, outs = (tok_v0, tok_v1), (out_v0, out_v1)
    isems, osems = (isem0, isem1), (osem0, osem1)
    wid = lax.axis_index("s") * NC + lax.axis_index("c")
    base = wid * UPW
    pltpu.sync_copy(scal_hbm, scal_v)
    _sincos_table(scal_v, table_v)

    def start_in(n, b):
        u = base + n
        jb = u >> 5
        ibb = u & 31
        pltpu.async_copy(
            tok_hbm.at[pl.ds(jb * 131072 + ibb * U_TOK, U_TOK)], toks[b],
            isems[b])

    def drain_in(b):
        pltpu.make_async_copy(
            tok_hbm.at[pl.ds(0, U_TOK)], toks[b], isems[b]).wait()

    def start_out(n, b):
        u = base + n
        jb = u >> 5
        ibb = u & 31
        for jr in range(8):
            pltpu.async_copy(
                outs[b].at[pl.ds(jr * 1024, 1024)],
                out_hbm.at[pl.ds((jb * 8 + jr) * 32768 + ibb * 1024, 1024)],
                osems[b])

    def drain_out(b):
        pltpu.make_async_copy(
            out_hbm.at[pl.ds(0, U_OUT)], outs[b], osems[b]).wait()

    def compute(b):
        tok_ref, out_ref = toks[b], outs[b]

        @plsc.parallel_loop(0, 256, unroll=8)
        def _vec(v):
            jr = v >> 5
            ibl = (v >> 3) & 3
            s = v & 7
            src = ibl * 1024 + jr * 128 + s * 16
            dst = jr * 1024 + ibl * 256 + s * 16
            t = tok_ref[pl.ds(src, 16)]
            out_ref[pl.ds(dst, 16)] = plsc.load_gather(table_v, [t])
            out_ref[pl.ds(dst + 128, 16)] = plsc.load_gather(
                table_v, [t + 16])

    start_in(0, 0)
    start_in(1, 1)

    @pl.loop(0, UPW - 1, step=2)
    def _g(g):
        for b in range(2):
            n = g + b
            drain_in(b)

            @pl.when(n >= 2)
            def _do(b=b):
                drain_out(b)

            compute(b)
            start_out(n, b)

            @pl.when(n + 2 < UPW)
            def _di(n=n, b=b):
                start_in(n + 2, b)

    drain_in(0)
    drain_out(0)
    compute(0)
    start_out(UPW - 1, 0)
    drain_out(1)
    drain_out(0)


def kernel(tokens, arc_A, arc_start, arc_stride):
    tok_flat = (
        tokens.T.reshape(JB, 8, IB, 128).transpose(0, 2, 1, 3).reshape(-1))
    sc = pl.kernel(
        _sc_body,
        out_type=jax.ShapeDtypeStruct((2 * N_TOK,), jnp.float32),
        mesh=plsc.VectorSubcoreMesh(core_axis_name="c", subcore_axis_name="s"),
        compiler_params=pltpu.CompilerParams(needs_layout_passes=False),
        scratch_types=[
            pltpu.VMEM((16,), jnp.float32),
            pltpu.VMEM((32,), jnp.float32),
            pltpu.VMEM((U_TOK,), jnp.int32),
            pltpu.VMEM((U_TOK,), jnp.int32),
            pltpu.VMEM((U_OUT,), jnp.float32),
            pltpu.VMEM((U_OUT,), jnp.float32),
            pltpu.SemaphoreType.DMA,
            pltpu.SemaphoreType.DMA,
            pltpu.SemaphoreType.DMA,
            pltpu.SemaphoreType.DMA,
        ],
    )
    scal = jnp.concatenate(
        [jnp.zeros((1,), jnp.float32), arc_A.reshape(1),
         arc_start.reshape(1), arc_stride.reshape(1),
         jnp.zeros((12,), jnp.float32)])
    out = sc(scal, tok_flat)
    return out.reshape(COLS, IB, 2, 128).transpose(1, 3, 0, 2).reshape(
        ROWS, COLS, 2)
